# baseline (speedup 1.0000x reference)
import functools

import jax
import jax.numpy as jnp
from jax import lax
from jax.experimental import pallas as pl
from jax.experimental.pallas import tpu as pltpu
from jax.experimental.pallas import tpu_sc as plsc

N_ATOMS = 1048576
N_TYPES = 119
N_PROPS = 64
N_SYSTEMS = 16384

TPAD = 119
NBINS = TPAD * N_SYSTEMS
NC = 2
NS = 16
NW = NC * NS
A_PER_W = N_ATOMS // NW
CHUNK = 2048
NBUF = 3
NCH = A_PER_W // CHUNK
ACC_PER_TILE = NBINS // NS


def _sc_histogram(keys, zeros_h, ones_h):
    mesh = plsc.VectorSubcoreMesh(
        core_axis_name="c", subcore_axis_name="s", num_cores=NC, num_subcores=NS
    )

    @functools.partial(
        pl.kernel,
        mesh=mesh,
        out_type=(
            jax.ShapeDtypeStruct((TPAD, N_SYSTEMS), jnp.float32),
            jax.ShapeDtypeStruct((TPAD, N_SYSTEMS), jnp.float32),
        ),
        scratch_types=[
            [pltpu.VMEM((CHUNK,), jnp.int32) for _ in range(NBUF)],
            pltpu.VMEM((CHUNK,), jnp.float32),
            pltpu.VMEM_SHARED((NBINS,), jnp.float32),
            pltpu.SemaphoreType.DMA,
            [pltpu.SemaphoreType.DMA for _ in range(NBUF)],
            [pltpu.SemaphoreType.DMA for _ in range(NBUF)],
        ],
    )
    def hist(keys_hbm, zeros_hbm, ones_hbm, out0, out1,
             kbufs, ones_v, acc, sem_z, sem_l, sem_s):
        c = lax.axis_index("c")
        s = lax.axis_index("s")
        w = c * NS + s

        zd = pltpu.async_copy(
            zeros_hbm, acc.at[pl.ds(s * ACC_PER_TILE, ACC_PER_TILE)], sem_z)
        pltpu.sync_copy(ones_hbm, ones_v)

        def start_load(k):
            b = k % NBUF
            base = w * A_PER_W + k * CHUNK
            return pltpu.async_copy(keys_hbm.at[pl.ds(base, CHUNK)],
                                    kbufs[b], sem_l[b])

        loads = [start_load(k) for k in range(NBUF - 1)] + [None] * (
            NCH - (NBUF - 1))
        zd.wait()
        plsc.subcore_barrier()

        scatters = [None] * NBUF
        for k in range(NCH):
            b = k % NBUF
            loads[k].wait()
            scatters[b] = pltpu.async_copy(
                ones_v, acc.at[kbufs[b]], sem_s[b], add=True)
            nxt = k + NBUF - 1
            if nxt < NCH:
                nb = nxt % NBUF
                if scatters[nb] is not None:
                    scatters[nb].wait()
                    scatters[nb] = None
                loads[nxt] = start_load(nxt)

        for d in scatters:
            if d is not None:
                d.wait()
        plsc.subcore_barrier()

        for i in range(8):
            r = i * NS + s

            @pl.when(r < TPAD)
            def _(r=r):
                row = acc.at[pl.ds(r * N_SYSTEMS, N_SYSTEMS)]

                @pl.when(c == 0)
                def _():
                    pltpu.sync_copy(row, out0.at[r])

                @pl.when(c == 1)
                def _():
                    pltpu.sync_copy(row, out1.at[r])

    return hist(keys, zeros_h, ones_h)


_BN = 8192
_NB = N_SYSTEMS // _BN


def _fused_body(c0_hbm, c1_hbm, w_ref, o_ref, lhs0, lhs1, sems):
    def start(j, b):
        sl = pl.ds(j * _BN, _BN)
        d0 = pltpu.make_async_copy(c0_hbm.at[:, sl], lhs0.at[b], sems.at[b, 0])
        d1 = pltpu.make_async_copy(c1_hbm.at[:, sl], lhs1.at[b], sems.at[b, 1])
        d0.start()
        d1.start()
        return d0, d1

    pend = start(0, 0)
    for j in range(_NB):
        b = j % 2
        pend[0].wait()
        pend[1].wait()
        if j + 1 < _NB:
            pend = start(j + 1, 1 - b)
        cnt = lhs0[b] + lhs1[b]
        o_ref[pl.ds(j * _BN, _BN), :] = lax.dot_general(
            cnt, w_ref[...], (((0,), (0,)), ((), ())),
            preferred_element_type=jnp.float32,
        )


def _tc_matmul(c0, c1, w_pad):
    return pl.pallas_call(
        _fused_body,
        in_specs=[
            pl.BlockSpec(memory_space=pltpu.HBM),
            pl.BlockSpec(memory_space=pltpu.HBM),
            pl.BlockSpec(memory_space=pltpu.VMEM),
        ],
        out_specs=pl.BlockSpec(memory_space=pltpu.VMEM),
        out_shape=jax.ShapeDtypeStruct((N_SYSTEMS, N_PROPS), jnp.float32),
        scratch_shapes=[
            pltpu.VMEM((2, TPAD, _BN), jnp.float32),
            pltpu.VMEM((2, TPAD, _BN), jnp.float32),
            pltpu.SemaphoreType.DMA((2, 2)),
        ],
    )(c0, c1, w_pad)


def kernel(weights, atom_types, system_ids, n_systems):
    del n_systems
    keys = atom_types * N_SYSTEMS + system_ids
    zeros_h = jnp.zeros((ACC_PER_TILE,), jnp.float32)
    ones_h = jnp.ones((CHUNK,), jnp.float32)
    c0, c1 = _sc_histogram(keys, zeros_h, ones_h)
    return _tc_matmul(c0, c1, weights)


# speedup vs baseline: 1.0259x; 1.0259x over previous
import functools

import jax
import jax.numpy as jnp
from jax import lax
from jax.experimental import pallas as pl
from jax.experimental.pallas import tpu as pltpu
from jax.experimental.pallas import tpu_sc as plsc

N_ATOMS = 1048576
N_TYPES = 119
N_PROPS = 64
N_SYSTEMS = 16384

TPAD = 119
NBINS = TPAD * N_SYSTEMS
NC = 2
NS = 16
NW = NC * NS
A_PER_W = N_ATOMS // NW
CHUNK = 1024
NBUF = 7
NCH = A_PER_W // CHUNK
ACC_PER_TILE = NBINS // NS


def _sc_histogram(keys, zeros_h, ones_h):
    mesh = plsc.VectorSubcoreMesh(
        core_axis_name="c", subcore_axis_name="s", num_cores=NC, num_subcores=NS
    )

    @functools.partial(
        pl.kernel,
        mesh=mesh,
        out_type=(
            jax.ShapeDtypeStruct((TPAD, N_SYSTEMS), jnp.float32),
            jax.ShapeDtypeStruct((TPAD, N_SYSTEMS), jnp.float32),
        ),
        scratch_types=[
            [pltpu.VMEM((CHUNK,), jnp.int32) for _ in range(NBUF)],
            pltpu.VMEM((CHUNK,), jnp.float32),
            pltpu.VMEM_SHARED((NBINS,), jnp.float32),
            pltpu.SemaphoreType.DMA,
            [pltpu.SemaphoreType.DMA for _ in range(NBUF)],
            [pltpu.SemaphoreType.DMA for _ in range(NBUF)],
        ],
    )
    def hist(keys_hbm, zeros_hbm, ones_hbm, out0, out1,
             kbufs, ones_v, acc, sem_z, sem_l, sem_s):
        c = lax.axis_index("c")
        s = lax.axis_index("s")
        w = c * NS + s

        zd = pltpu.async_copy(
            zeros_hbm, acc.at[pl.ds(s * ACC_PER_TILE, ACC_PER_TILE)], sem_z)
        pltpu.sync_copy(ones_hbm, ones_v)

        def start_load(k):
            b = k % NBUF
            base = w * A_PER_W + k * CHUNK
            return pltpu.async_copy(keys_hbm.at[pl.ds(base, CHUNK)],
                                    kbufs[b], sem_l[b])

        loads = [start_load(k) for k in range(NBUF - 1)] + [None] * (
            NCH - (NBUF - 1))
        zd.wait()
        plsc.subcore_barrier()

        scatters = [None] * NBUF
        for k in range(NCH):
            b = k % NBUF
            loads[k].wait()
            scatters[b] = pltpu.async_copy(
                ones_v, acc.at[kbufs[b]], sem_s[b], add=True)
            nxt = k + NBUF - 1
            if nxt < NCH:
                nb = nxt % NBUF
                if scatters[nb] is not None:
                    scatters[nb].wait()
                    scatters[nb] = None
                loads[nxt] = start_load(nxt)

        for d in scatters:
            if d is not None:
                d.wait()
        plsc.subcore_barrier()

        for i in range(8):
            r = i * NS + s

            @pl.when(r < TPAD)
            def _(r=r):
                row = acc.at[pl.ds(r * N_SYSTEMS, N_SYSTEMS)]

                @pl.when(c == 0)
                def _():
                    pltpu.sync_copy(row, out0.at[r])

                @pl.when(c == 1)
                def _():
                    pltpu.sync_copy(row, out1.at[r])

    return hist(keys, zeros_h, ones_h)


_BN = 8192
_NB = N_SYSTEMS // _BN


def _fused_body(c0_hbm, c1_hbm, w_ref, o_ref, lhs0, lhs1, sems):
    def start(j, b):
        sl = pl.ds(j * _BN, _BN)
        d0 = pltpu.make_async_copy(c0_hbm.at[:, sl], lhs0.at[b], sems.at[b, 0])
        d1 = pltpu.make_async_copy(c1_hbm.at[:, sl], lhs1.at[b], sems.at[b, 1])
        d0.start()
        d1.start()
        return d0, d1

    pend = start(0, 0)
    for j in range(_NB):
        b = j % 2
        pend[0].wait()
        pend[1].wait()
        if j + 1 < _NB:
            pend = start(j + 1, 1 - b)
        cnt = lhs0[b] + lhs1[b]
        o_ref[pl.ds(j * _BN, _BN), :] = lax.dot_general(
            cnt, w_ref[...], (((0,), (0,)), ((), ())),
            preferred_element_type=jnp.float32,
        )


def _tc_matmul(c0, c1, w_pad):
    return pl.pallas_call(
        _fused_body,
        in_specs=[
            pl.BlockSpec(memory_space=pltpu.HBM),
            pl.BlockSpec(memory_space=pltpu.HBM),
            pl.BlockSpec(memory_space=pltpu.VMEM),
        ],
        out_specs=pl.BlockSpec(memory_space=pltpu.VMEM),
        out_shape=jax.ShapeDtypeStruct((N_SYSTEMS, N_PROPS), jnp.float32),
        scratch_shapes=[
            pltpu.VMEM((2, TPAD, _BN), jnp.float32),
            pltpu.VMEM((2, TPAD, _BN), jnp.float32),
            pltpu.SemaphoreType.DMA((2, 2)),
        ],
    )(c0, c1, w_pad)


def kernel(weights, atom_types, system_ids, n_systems):
    del n_systems
    keys = atom_types * N_SYSTEMS + system_ids
    zeros_h = jnp.zeros((ACC_PER_TILE,), jnp.float32)
    ones_h = jnp.ones((CHUNK,), jnp.float32)
    c0, c1 = _sc_histogram(keys, zeros_h, ones_h)
    return _tc_matmul(c0, c1, weights)


# speedup vs baseline: 1.1466x; 1.1177x over previous
import functools

import jax
import jax.numpy as jnp
from jax import lax
from jax.experimental import pallas as pl
from jax.experimental.pallas import tpu as pltpu
from jax.experimental.pallas import tpu_sc as plsc

N_ATOMS = 1048576
N_TYPES = 119
N_PROPS = 64
N_SYSTEMS = 16384

TPAD = 119
NBINS = TPAD * N_SYSTEMS
NC = 2
NS = 16
NW = NC * NS
A_PER_W = N_ATOMS // NW
CHUNK = 1024
NBUF = 7
NCH = A_PER_W // CHUNK
ACC_PER_TILE = NBINS // NS


def _sc_histogram(keys, zeros_h, ones_h):
    mesh = plsc.VectorSubcoreMesh(
        core_axis_name="c", subcore_axis_name="s", num_cores=NC, num_subcores=NS
    )

    @functools.partial(
        pl.kernel,
        mesh=mesh,
        out_type=(
            jax.ShapeDtypeStruct((TPAD, N_SYSTEMS), jnp.float32),
            jax.ShapeDtypeStruct((TPAD, N_SYSTEMS), jnp.float32),
        ),
        scratch_types=[
            [pltpu.VMEM((CHUNK,), jnp.int32) for _ in range(NBUF)],
            pltpu.VMEM((CHUNK,), jnp.float32),
            pltpu.VMEM_SHARED((NBINS,), jnp.float32),
            pltpu.SemaphoreType.DMA,
            [pltpu.SemaphoreType.DMA for _ in range(NBUF)],
            [pltpu.SemaphoreType.DMA for _ in range(NBUF)],
        ],
    )
    def hist(keys_hbm, zeros_hbm, ones_hbm, out0, out1,
             kbufs, ones_v, acc, sem_z, sem_l, sem_s):
        c = lax.axis_index("c")
        s = lax.axis_index("s")
        w = c * NS + s

        zd = pltpu.async_copy(
            zeros_hbm, acc.at[pl.ds(s * ACC_PER_TILE, ACC_PER_TILE)], sem_z)
        pltpu.sync_copy(ones_hbm, ones_v)

        def start_load(k):
            b = k % NBUF
            base = w * A_PER_W + k * CHUNK
            return pltpu.async_copy(keys_hbm.at[pl.ds(base, CHUNK)],
                                    kbufs[b], sem_l[b])

        loads = [start_load(k) for k in range(NBUF - 1)] + [None] * (
            NCH - (NBUF - 1))
        zd.wait()
        plsc.subcore_barrier()

        scatters = [None] * NBUF
        for k in range(NCH):
            b = k % NBUF
            loads[k].wait()
            scatters[b] = pltpu.async_copy(
                ones_v, acc.at[kbufs[b]], sem_s[b], add=True)
            nxt = k + NBUF - 1
            if nxt < NCH:
                nb = nxt % NBUF
                if scatters[nb] is not None:
                    scatters[nb].wait()
                    scatters[nb] = None
                loads[nxt] = start_load(nxt)

        for d in scatters:
            if d is not None:
                d.wait()
        plsc.subcore_barrier()

        for i in range(8):
            r = i * NS + s

            @pl.when(r < TPAD)
            def _(r=r):
                row = acc.at[pl.ds(r * N_SYSTEMS, N_SYSTEMS)]

                @pl.when(c == 0)
                def _():
                    pltpu.sync_copy(row, out0.at[r])

                @pl.when(c == 1)
                def _():
                    pltpu.sync_copy(row, out1.at[r])

    return hist(keys, zeros_h, ones_h)


_BN = 8192
_NB = N_SYSTEMS // _BN


def _fused_body(c0_hbm, c1_hbm, w_ref, o_ref, lhs0, lhs1, sems):
    def start(j, b):
        sl = pl.ds(j * _BN, _BN)
        d0 = pltpu.make_async_copy(c0_hbm.at[:, sl], lhs0.at[b], sems.at[b, 0])
        d1 = pltpu.make_async_copy(c1_hbm.at[:, sl], lhs1.at[b], sems.at[b, 1])
        d0.start()
        d1.start()
        return d0, d1

    pend = start(0, 0)
    for j in range(_NB):
        b = j % 2
        pend[0].wait()
        pend[1].wait()
        if j + 1 < _NB:
            pend = start(j + 1, 1 - b)
        cnt = lhs0[b] + lhs1[b]
        o_ref[:, pl.ds(j * _BN, _BN)] = lax.dot_general(
            w_ref[...], cnt, (((0,), (0,)), ((), ())),
            preferred_element_type=jnp.float32,
        )


def _tc_matmul(c0, c1, w_pad):
    return pl.pallas_call(
        _fused_body,
        in_specs=[
            pl.BlockSpec(memory_space=pltpu.HBM),
            pl.BlockSpec(memory_space=pltpu.HBM),
            pl.BlockSpec(memory_space=pltpu.VMEM),
        ],
        out_specs=pl.BlockSpec(memory_space=pltpu.VMEM),
        out_shape=jax.ShapeDtypeStruct((N_PROPS, N_SYSTEMS), jnp.float32),
        scratch_shapes=[
            pltpu.VMEM((2, TPAD, _BN), jnp.float32),
            pltpu.VMEM((2, TPAD, _BN), jnp.float32),
            pltpu.SemaphoreType.DMA((2, 2)),
        ],
    )(c0, c1, w_pad)


def kernel(weights, atom_types, system_ids, n_systems):
    del n_systems
    keys = atom_types * N_SYSTEMS + system_ids
    zeros_h = jnp.zeros((ACC_PER_TILE,), jnp.float32)
    ones_h = jnp.ones((CHUNK,), jnp.float32)
    c0, c1 = _sc_histogram(keys, zeros_h, ones_h)
    return _tc_matmul(c0, c1, weights).T
